# baseline (speedup 1.0000x reference)
import functools

import jax
import jax.numpy as jnp
from jax import lax
from jax.experimental import pallas as pl
from jax.experimental.pallas import tpu as pltpu
from jax.experimental.pallas import tpu_sc as plsc

B = 4096
EMB = 128
N_DENSE = 13
BM = 256
NBLK = B // BM

_NC = 2
_NS = 16
_NW = _NC * _NS


def _sc_gather(emb, idx):
    d = emb.shape[1]
    bpw = B // _NW

    mesh = plsc.VectorSubcoreMesh(core_axis_name="c", subcore_axis_name="s")

    @functools.partial(
        pl.kernel,
        mesh=mesh,
        out_type=jax.ShapeDtypeStruct((B, d), jnp.float32),
        scratch_types=[
            pltpu.VMEM((bpw,), jnp.int32),
            pltpu.VMEM((bpw, d), jnp.float32),
            pltpu.SemaphoreType.DMA,
        ],
    )
    def gk(emb_hbm, idx_hbm, out_hbm, idx_v, rows_v, sem):
        wid = lax.axis_index("s") * _NC + lax.axis_index("c")
        base = wid * bpw
        pltpu.sync_copy(idx_hbm.at[pl.ds(base, bpw)], idx_v)
        pltpu.async_copy(emb_hbm.at[idx_v], rows_v, sem).wait()
        pltpu.sync_copy(rows_v, out_hbm.at[pl.ds(base, bpw)])

    return gk(emb, idx)


def _dense_body(e_ref, dn_ref, w0e_ref, w0d_ref, b0_ref, w1_ref, b1_ref,
                w2_ref, b2_ref, wo_ref, bo_ref, fwe_ref, fwd_ref, fb_ref,
                out_ref):
    i = pl.program_id(0)
    dn_t = (((1,), (1,)), ((), ()))

    eb = e_ref[pl.ds(i * BM, BM), :]
    db = dn_ref[pl.ds(i * BM, BM), :]

    h = lax.dot_general(eb, w0e_ref[...], dn_t, preferred_element_type=jnp.float32)
    h += lax.dot_general(db, w0d_ref[...], dn_t, preferred_element_type=jnp.float32)
    h = jnp.maximum(h + b0_ref[...], 0.0)
    h = lax.dot_general(h, w1_ref[...], dn_t, preferred_element_type=jnp.float32)
    h = jnp.maximum(h + b1_ref[...], 0.0)
    h = lax.dot_general(h, w2_ref[...], dn_t, preferred_element_type=jnp.float32)
    h = jnp.maximum(h + b2_ref[...], 0.0)
    a = jnp.sum(h * wo_ref[...], axis=1, keepdims=True)
    a += bo_ref[0, 0]

    brow = lax.dot_general(fwe_ref[...], e_ref[...], dn_t,
                           preferred_element_type=jnp.float32)
    brow += lax.dot_general(fwd_ref[...], dn_ref[...], dn_t,
                            preferred_element_type=jnp.float32)
    brow += fb_ref[0, 0]

    out_ref[...] = jax.nn.sigmoid(a + brow)


def _full(shape):
    return pl.BlockSpec(shape, lambda i: (0,) * len(shape))


def _dense_call(e, dense, w0e, w0d, b0, w1, b1, w2, b2, wo, bo, fwe, fwd, fb):
    return pl.pallas_call(
        _dense_body,
        grid=(NBLK,),
        in_specs=[
            _full((B, EMB)),
            _full((B, N_DENSE)),
            _full(w0e.shape),
            _full(w0d.shape),
            _full(b0.shape),
            _full(w1.shape),
            _full(b1.shape),
            _full(w2.shape),
            _full(b2.shape),
            _full(wo.shape),
            _full(bo.shape),
            _full(fwe.shape),
            _full(fwd.shape),
            _full(fb.shape),
        ],
        out_specs=pl.BlockSpec((BM, B), lambda i: (i, 0)),
        out_shape=jax.ShapeDtypeStruct((B, B), jnp.float32),
    )(e, dense, w0e, w0d, b0, w1, b1, w2, b2, wo, bo, fwe, fwd, fb)


def kernel(sparse_inputs, dense_inputs, emb, fm_w, fm_b, w0, b0, w1, b1, w2,
           b2, wo, bo):
    idx = sparse_inputs.reshape(-1).astype(jnp.int32)
    e = _sc_gather(emb, idx)

    w0e = w0[:, :EMB]
    w0d = w0[:, EMB:]
    fwe = fm_w[:, :EMB]
    fwd = fm_w[:, EMB:]
    return _dense_call(
        e, dense_inputs, w0e, w0d, b0.reshape(1, -1), w1, b1.reshape(1, -1),
        w2, b2.reshape(1, -1), wo, bo.reshape(1, 1), fwe, fwd,
        fm_b.reshape(1, 1))


# speedup vs baseline: 1.1382x; 1.1382x over previous
import functools

import jax
import jax.numpy as jnp
from jax import lax
from jax.experimental import pallas as pl
from jax.experimental.pallas import tpu as pltpu
from jax.experimental.pallas import tpu_sc as plsc

B = 4096
EMB = 128
N_DENSE = 13
BM = 512
NBLK = B // BM

_NC = 2
_NS = 16
_NW = _NC * _NS


def _sc_gather(emb, idx):
    d = emb.shape[1]
    bpw = B // _NW

    mesh = plsc.VectorSubcoreMesh(core_axis_name="c", subcore_axis_name="s")

    @functools.partial(
        pl.kernel,
        mesh=mesh,
        out_type=jax.ShapeDtypeStruct((B, d), jnp.float32),
        scratch_types=[
            pltpu.VMEM((bpw,), jnp.int32),
            pltpu.VMEM((bpw, d), jnp.float32),
            pltpu.SemaphoreType.DMA,
        ],
    )
    def gk(emb_hbm, idx_hbm, out_hbm, idx_v, rows_v, sem):
        wid = lax.axis_index("s") * _NC + lax.axis_index("c")
        base = wid * bpw
        pltpu.sync_copy(idx_hbm.at[pl.ds(base, bpw)], idx_v)
        pltpu.async_copy(emb_hbm.at[idx_v], rows_v, sem).wait()
        pltpu.sync_copy(rows_v, out_hbm.at[pl.ds(base, bpw)])

    return gk(emb, idx)


def _dense_body(e_ref, dn_ref, w0e_ref, w0d_ref, b0_ref, w1_ref, b1_ref,
                w2_ref, b2_ref, wo_ref, bo_ref, fwe_ref, fwd_ref, fb_ref,
                out_ref, brow_s):
    i = pl.program_id(0)
    dn_t = (((1,), (1,)), ((), ()))

    @pl.when(i == 0)
    def _():
        brow = lax.dot_general(fwe_ref[...], e_ref[...], dn_t,
                               preferred_element_type=jnp.float32)
        brow += lax.dot_general(fwd_ref[...], dn_ref[...], dn_t,
                                preferred_element_type=jnp.float32)
        brow_s[...] = brow + fb_ref[0, 0]

    eb = e_ref[pl.ds(i * BM, BM), :]
    db = dn_ref[pl.ds(i * BM, BM), :]

    h = lax.dot_general(eb, w0e_ref[...], dn_t, preferred_element_type=jnp.float32)
    h += lax.dot_general(db, w0d_ref[...], dn_t, preferred_element_type=jnp.float32)
    h = jnp.maximum(h + b0_ref[...], 0.0)
    h = lax.dot_general(h, w1_ref[...], dn_t, preferred_element_type=jnp.float32)
    h = jnp.maximum(h + b1_ref[...], 0.0)
    h = lax.dot_general(h, w2_ref[...], dn_t, preferred_element_type=jnp.float32)
    h = jnp.maximum(h + b2_ref[...], 0.0)
    a = jnp.sum(h * wo_ref[...], axis=1, keepdims=True)
    a += bo_ref[0, 0]

    out_ref[...] = 0.5 * jnp.tanh(0.5 * (a + brow_s[...])) + 0.5


def _full(shape):
    return pl.BlockSpec(shape, lambda i: (0,) * len(shape))


def _dense_call(e, dense, w0e, w0d, b0, w1, b1, w2, b2, wo, bo, fwe, fwd, fb):
    return pl.pallas_call(
        _dense_body,
        grid=(NBLK,),
        in_specs=[
            _full((B, EMB)),
            _full((B, N_DENSE)),
            _full(w0e.shape),
            _full(w0d.shape),
            _full(b0.shape),
            _full(w1.shape),
            _full(b1.shape),
            _full(w2.shape),
            _full(b2.shape),
            _full(wo.shape),
            _full(bo.shape),
            _full(fwe.shape),
            _full(fwd.shape),
            _full(fb.shape),
        ],
        out_specs=pl.BlockSpec((BM, B), lambda i: (i, 0)),
        out_shape=jax.ShapeDtypeStruct((B, B), jnp.float32),
        scratch_shapes=[pltpu.VMEM((1, B), jnp.float32)],
    )(e, dense, w0e, w0d, b0, w1, b1, w2, b2, wo, bo, fwe, fwd, fb)


def kernel(sparse_inputs, dense_inputs, emb, fm_w, fm_b, w0, b0, w1, b1, w2,
           b2, wo, bo):
    idx = sparse_inputs.reshape(-1).astype(jnp.int32)
    e = _sc_gather(emb, idx)

    w0e = w0[:, :EMB]
    w0d = w0[:, EMB:]
    fwe = fm_w[:, :EMB]
    fwd = fm_w[:, EMB:]
    return _dense_call(
        e, dense_inputs, w0e, w0d, b0.reshape(1, -1), w1, b1.reshape(1, -1),
        w2, b2.reshape(1, -1), wo, bo.reshape(1, 1), fwe, fwd,
        fm_b.reshape(1, 1))


# speedup vs baseline: 1.1450x; 1.0060x over previous
import functools

import jax
import jax.numpy as jnp
from jax import lax
from jax.experimental import pallas as pl
from jax.experimental.pallas import tpu as pltpu
from jax.experimental.pallas import tpu_sc as plsc

B = 4096
EMB = 128
N_DENSE = 13
BM = 512
NBLK = B // BM

_NC = 2
_NS = 16
_NW = _NC * _NS


def _sc_gather(emb, idx):
    d = emb.shape[1]
    bpw = B // _NW

    mesh = plsc.VectorSubcoreMesh(core_axis_name="c", subcore_axis_name="s")

    half = bpw // 2

    @functools.partial(
        pl.kernel,
        mesh=mesh,
        out_type=jax.ShapeDtypeStruct((B, d), jnp.float32),
        scratch_types=[
            pltpu.VMEM((half,), jnp.int32),
            pltpu.VMEM((half,), jnp.int32),
            pltpu.VMEM((half, d), jnp.float32),
            pltpu.VMEM((half, d), jnp.float32),
            pltpu.SemaphoreType.DMA,
            pltpu.SemaphoreType.DMA,
            pltpu.SemaphoreType.DMA,
            pltpu.SemaphoreType.DMA,
        ],
    )
    def gk(emb_hbm, idx_hbm, out_hbm, i0, i1, r0, r1, s0, s1, s2, s3):
        wid = lax.axis_index("s") * _NC + lax.axis_index("c")
        base = wid * bpw
        pltpu.sync_copy(idx_hbm.at[pl.ds(base, half)], i0)
        g0 = pltpu.async_copy(emb_hbm.at[i0], r0, s0)
        pltpu.sync_copy(idx_hbm.at[pl.ds(base + half, half)], i1)
        g1 = pltpu.async_copy(emb_hbm.at[i1], r1, s1)
        g0.wait()
        w0 = pltpu.async_copy(r0, out_hbm.at[pl.ds(base, half)], s2)
        g1.wait()
        w1 = pltpu.async_copy(r1, out_hbm.at[pl.ds(base + half, half)], s3)
        w0.wait()
        w1.wait()

    return gk(emb, idx)


def _dense_body(e_ref, dn_ref, w0e_ref, w0d_ref, b0_ref, w1_ref, b1_ref,
                w2_ref, b2_ref, wo_ref, bo_ref, fwe_ref, fwd_ref, fb_ref,
                out_ref, brow_s):
    i = pl.program_id(0)
    dn_t = (((1,), (1,)), ((), ()))

    @pl.when(i == 0)
    def _():
        brow = lax.dot_general(fwe_ref[...], e_ref[...], dn_t,
                               preferred_element_type=jnp.float32)
        brow += lax.dot_general(fwd_ref[...], dn_ref[...], dn_t,
                                preferred_element_type=jnp.float32)
        brow_s[...] = brow + fb_ref[0, 0]

    eb = e_ref[pl.ds(i * BM, BM), :]
    db = dn_ref[pl.ds(i * BM, BM), :]

    h = lax.dot_general(eb, w0e_ref[...], dn_t, preferred_element_type=jnp.float32)
    h += lax.dot_general(db, w0d_ref[...], dn_t, preferred_element_type=jnp.float32)
    h = jnp.maximum(h + b0_ref[...], 0.0)
    h = lax.dot_general(h, w1_ref[...], dn_t, preferred_element_type=jnp.float32)
    h = jnp.maximum(h + b1_ref[...], 0.0)
    h = lax.dot_general(h, w2_ref[...], dn_t, preferred_element_type=jnp.float32)
    h = jnp.maximum(h + b2_ref[...], 0.0)
    a = jnp.sum(h * wo_ref[...], axis=1, keepdims=True)
    a += bo_ref[0, 0]

    out_ref[...] = 0.5 * jnp.tanh(0.5 * (a + brow_s[...])) + 0.5


def _full(shape):
    return pl.BlockSpec(shape, lambda i: (0,) * len(shape))


def _dense_call(e, dense, w0e, w0d, b0, w1, b1, w2, b2, wo, bo, fwe, fwd, fb):
    return pl.pallas_call(
        _dense_body,
        grid=(NBLK,),
        in_specs=[
            _full((B, EMB)),
            _full((B, N_DENSE)),
            _full(w0e.shape),
            _full(w0d.shape),
            _full(b0.shape),
            _full(w1.shape),
            _full(b1.shape),
            _full(w2.shape),
            _full(b2.shape),
            _full(wo.shape),
            _full(bo.shape),
            _full(fwe.shape),
            _full(fwd.shape),
            _full(fb.shape),
        ],
        out_specs=pl.BlockSpec((BM, B), lambda i: (i, 0)),
        out_shape=jax.ShapeDtypeStruct((B, B), jnp.float32),
        scratch_shapes=[pltpu.VMEM((1, B), jnp.float32)],
    )(e, dense, w0e, w0d, b0, w1, b1, w2, b2, wo, bo, fwe, fwd, fb)


def kernel(sparse_inputs, dense_inputs, emb, fm_w, fm_b, w0, b0, w1, b1, w2,
           b2, wo, bo):
    idx = sparse_inputs.reshape(-1).astype(jnp.int32)
    e = _sc_gather(emb, idx)

    w0e = w0[:, :EMB]
    w0d = w0[:, EMB:]
    fwe = fm_w[:, :EMB]
    fwd = fm_w[:, EMB:]
    return _dense_call(
        e, dense_inputs, w0e, w0d, b0.reshape(1, -1), w1, b1.reshape(1, -1),
        w2, b2.reshape(1, -1), wo, bo.reshape(1, 1), fwe, fwd,
        fm_b.reshape(1, 1))
